# baseline (speedup 1.0000x reference)
import jax
import jax.numpy as jnp
from jax import lax
from jax.experimental import pallas as pl
from jax.experimental.pallas import tpu as pltpu
from jax.experimental.pallas import tpu_sc as plsc

B, N = 128, 32768
NW = 32
ROWS_PER_W = B // NW
Q = 4
QLEN = N // Q
GROUPS = QLEN // 16
NEG_MASK = -1000000.0
NEG_COL0 = -100000.0
I32_MAX = 2147483647


def _sc_body(curr_hbm, orig_hbm, mask_hbm, frac_hbm, sel_hbm,
             curr_v, orig_v, mask_v, sel_v):
    cid = lax.axis_index("c")
    sid = lax.axis_index("s")
    wid = sid * 2 + cid
    lanes = lax.iota(jnp.int32, 16)
    sel_acc = jnp.zeros((16,), jnp.int32)

    for r in range(ROWS_PER_W):
        row = wid * ROWS_PER_W + r
        pltpu.sync_copy(curr_hbm.at[row], curr_v)
        pltpu.sync_copy(orig_hbm.at[row], orig_v)
        pltpu.sync_copy(mask_hbm.at[row], mask_v)

        init = []
        for _ in range(Q):
            init.append(jnp.full((16,), -3.0e38, jnp.float32))
            init.append(jnp.zeros((16,), jnp.int32))

        def body(p, carry):
            carry = list(carry)
            words = mask_v[pl.ds(p * 16, 16)]
            for j in range(Q):
                off = p * 16 + j * QLEN
                c = curr_v[pl.ds(off, 16)]
                o = orig_v[pl.ds(off, 16)]
                f = c / o
                mb = lax.shift_right_logical(words, 8 * j) & 0xFF
                f = jnp.where(mb != 0, NEG_MASK, f)
                idx = off + lanes
                if j == 0:
                    f = jnp.where(idx == 0, NEG_COL0, f)
                curr_v[pl.ds(off, 16)] = f
                rm, ri = carry[2 * j], carry[2 * j + 1]
                upd = f > rm
                carry[2 * j] = jnp.where(upd, f, rm)
                carry[2 * j + 1] = jnp.where(upd, idx, ri)
            return tuple(carry)

        fin = lax.fori_loop(0, GROUPS, body, tuple(init))
        pltpu.sync_copy(curr_v, frac_hbm.at[row])

        m01 = jnp.where(fin[0] >= fin[2], fin[0], fin[2])
        i01 = jnp.where(fin[0] >= fin[2], fin[1], fin[3])
        m23 = jnp.where(fin[4] >= fin[6], fin[4], fin[6])
        i23 = jnp.where(fin[4] >= fin[6], fin[5], fin[7])
        m = jnp.where(m01 >= m23, m01, m23)
        i = jnp.where(m01 >= m23, i01, i23)
        for sh in (8, 4, 2, 1):
            part = lanes ^ sh
            pm = m.at[part].get(mode="promise_in_bounds")
            pi = i.at[part].get(mode="promise_in_bounds")
            better = (pm > m) | ((pm == m) & (pi < i))
            m = jnp.where(better, pm, m)
            i = jnp.where(better, pi, i)
        sel_acc = jnp.where(lanes == r, i, sel_acc)

    sel_v[...] = sel_acc
    pltpu.sync_copy(sel_v, sel_hbm.at[wid])


_sc_call = pl.kernel(
    _sc_body,
    out_type=[
        jax.ShapeDtypeStruct((B, N), jnp.float32),
        jax.ShapeDtypeStruct((NW, 16), jnp.int32),
    ],
    scratch_types=[
        pltpu.VMEM((N,), jnp.float32),
        pltpu.VMEM((N,), jnp.float32),
        pltpu.VMEM((QLEN,), jnp.int32),
        pltpu.VMEM((16,), jnp.int32),
    ],
    mesh=plsc.VectorSubcoreMesh(core_axis_name="c", subcore_axis_name="s"),
)


def kernel(curr_budget, orig_budget, mask):
    m8 = mask.astype(jnp.uint8).reshape(B, Q, QLEN)
    m8 = jnp.transpose(m8, (0, 2, 1))
    m32 = lax.bitcast_convert_type(m8, jnp.int32)
    frac, sel_raw = _sc_call(curr_budget, orig_budget, m32)
    selected = sel_raw[:, :ROWS_PER_W].reshape(B, 1)
    return frac, selected
